# baseline (speedup 1.0000x reference)
import functools

import jax
import jax.numpy as jnp
from jax import lax
from jax.experimental import pallas as pl
from jax.experimental.pallas import tpu as pltpu
from jax.experimental.pallas import tpu_sc as plsc

B = 16384
D = 32
L = 16
NC = 2
NS = 16
NW = NC * NS
BPW = B // NW
CHUNK = 128
NCHUNK = BPW // CHUNK
N = 1_000_000
NPAD = 1_000_064
TC_PER_GRP = 13
GRP = 601
NGRP = 4 * GRP
GRP_W = TC_PER_GRP * CHUNK
PER_WORKER = (NGRP + NW - 1) // NW


NBUF = 4


def _detile_body(utab_hbm, itab_hbm, uflat_hbm, iflat_hbm,
                 slab_u0, slab_i0, slab_u1, slab_i1,
                 slab_u2, slab_i2, slab_u3, slab_i3,
                 rd_u, rd_i, wr_u, wr_i):
    wid = lax.axis_index("s") * NC + lax.axis_index("c")
    slabs = ((slab_u0, slab_i0), (slab_u1, slab_i1),
             (slab_u2, slab_i2), (slab_u3, slab_i3))
    dummy_src = utab_hbm.at[pl.ds(0, 8), pl.ds(0, GRP_W)]

    def step(k, slot):
        su, si = slabs[slot]
        gid = wid + k * NW

        @pl.when(gid < NGRP)
        def _():
            tr = gid // GRP
            tc0 = (gid % GRP) * TC_PER_GRP

            @pl.when(k >= NBUF)
            def _():
                pltpu.make_async_copy(dummy_src, su, wr_u).wait()
                pltpu.make_async_copy(dummy_src, si, wr_i).wait()

            cu = pltpu.async_copy(
                utab_hbm.at[pl.ds(tr * 8, 8), pl.ds(tc0 * CHUNK, GRP_W)],
                su, rd_u)
            ci = pltpu.async_copy(
                itab_hbm.at[pl.ds(tr * 8, 8), pl.ds(tc0 * CHUNK, GRP_W)],
                si, rd_i)
            cu.wait()
            ci.wait()
            for dr in range(8):
                off = (tr * 8 + dr) * NPAD + tc0 * CHUNK
                pltpu.async_copy(su.at[dr], uflat_hbm.at[pl.ds(off, GRP_W)], wr_u)
                pltpu.async_copy(si.at[dr], iflat_hbm.at[pl.ds(off, GRP_W)], wr_i)

    def body(kk, carry):
        for b2 in range(NBUF):
            step(NBUF * kk + b2, b2)
        return carry

    lax.fori_loop(0, (PER_WORKER + NBUF - 1) // NBUF, body, 0)

    ng = (NGRP - wid + NW - 1) // NW
    for b2 in range(NBUF):
        su, si = slabs[b2]

        @pl.when(ng >= b2 + 1)
        def _(su=su, si=si):
            pltpu.make_async_copy(dummy_src, su, wr_u).wait()
            pltpu.make_async_copy(dummy_src, si, wr_i).wait()


def _gather_body(uidx_hbm, iidx_hbm, w_hbm, b_hbm, uflat_hbm, iflat_hbm,
                 out_hbm, uidx_v, iidx_v, uaddr_v, iaddr_v, urows_v, irows_v,
                 w_v, b_v, out_v, sem_u, sem_i):
    wid = lax.axis_index("s") * NC + lax.axis_index("c")
    base = wid * BPW

    pltpu.sync_copy(uidx_hbm.at[wid], uidx_v)
    pltpu.sync_copy(iidx_hbm.at[wid], iidx_v)
    pltpu.sync_copy(w_hbm, w_v)
    pltpu.sync_copy(b_hbm, b_v)

    for c in range(NCHUNK):
        for j in range(CHUNK // L):
            sl = pl.ds(c * CHUNK + j * L, L)
            uiv = uidx_v[c, pl.ds(j * L, L)]
            iiv = iidx_v[c, pl.ds(j * L, L)]
            for d in range(D):
                uaddr_v[d, sl] = uiv + d * NPAD
                iaddr_v[d, sl] = iiv + d * NPAD

    copies = []
    for d in range(D):
        copies.append(pltpu.async_copy(
            uflat_hbm.at[uaddr_v.at[d]], urows_v.at[d], sem_u))
        copies.append(pltpu.async_copy(
            iflat_hbm.at[iaddr_v.at[d]], irows_v.at[d], sem_i))
    for cp in copies:
        cp.wait()

    bvec = b_v[...]
    wvecs = [w_v[d] for d in range(D)]

    def body(g, carry):
        acc = bvec
        for d in range(D):
            uu = urows_v[d, pl.ds(g * L, L)]
            ii = irows_v[d, pl.ds(g * L, L)]
            acc = acc + uu * ii * wvecs[d]
        out_v[pl.ds(g * L, L)] = acc
        return carry

    lax.fori_loop(0, BPW // L, body, 0)
    pltpu.sync_copy(out_v, out_hbm.at[pl.ds(base, BPW)])


def _detile_call(utabT, itabT):
    mesh = plsc.VectorSubcoreMesh(core_axis_name="c", subcore_axis_name="s")
    kern = functools.partial(
        pl.kernel,
        mesh=mesh,
        compiler_params=pltpu.CompilerParams(
            needs_layout_passes=False, use_tc_tiling_on_sc=True),
        out_type=(jax.ShapeDtypeStruct((D * NPAD,), jnp.float32),
                  jax.ShapeDtypeStruct((D * NPAD,), jnp.float32)),
        scratch_types=[
            pltpu.VMEM((8, GRP_W), jnp.float32),
            pltpu.VMEM((8, GRP_W), jnp.float32),
            pltpu.VMEM((8, GRP_W), jnp.float32),
            pltpu.VMEM((8, GRP_W), jnp.float32),
            pltpu.VMEM((8, GRP_W), jnp.float32),
            pltpu.VMEM((8, GRP_W), jnp.float32),
            pltpu.VMEM((8, GRP_W), jnp.float32),
            pltpu.VMEM((8, GRP_W), jnp.float32),
            pltpu.SemaphoreType.DMA,
            pltpu.SemaphoreType.DMA,
            pltpu.SemaphoreType.DMA,
            pltpu.SemaphoreType.DMA,
        ],
    )(_detile_body)
    return kern(utabT, itabT)


def _gather_call(uidx3, iidx3, wb, b16, uflat, iflat):
    mesh = plsc.VectorSubcoreMesh(core_axis_name="c", subcore_axis_name="s")
    kern = functools.partial(
        pl.kernel,
        mesh=mesh,
        compiler_params=pltpu.CompilerParams(
            needs_layout_passes=False, use_tc_tiling_on_sc=False),
        out_type=jax.ShapeDtypeStruct((B,), jnp.float32),
        scratch_types=[
            pltpu.VMEM((NCHUNK, CHUNK), jnp.int32),
            pltpu.VMEM((NCHUNK, CHUNK), jnp.int32),
            pltpu.VMEM((D, BPW), jnp.int32),
            pltpu.VMEM((D, BPW), jnp.int32),
            pltpu.VMEM((D, BPW), jnp.float32),
            pltpu.VMEM((D, BPW), jnp.float32),
            pltpu.VMEM((D, L), jnp.float32),
            pltpu.VMEM((L,), jnp.float32),
            pltpu.VMEM((BPW,), jnp.float32),
            pltpu.SemaphoreType.DMA,
            pltpu.SemaphoreType.DMA,
        ],
    )(_gather_body)
    return kern(uidx3, iidx3, wb, b16, uflat, iflat)


def kernel(user_indices, item_indices, ratings, user_table, item_table, W, b):
    del ratings
    uidx3 = user_indices.reshape(NW, NCHUNK, CHUNK)
    iidx3 = item_indices.reshape(NW, NCHUNK, CHUNK)
    wb = jnp.broadcast_to(W, (D, L))
    b16 = jnp.broadcast_to(b, (L,))
    uflat, iflat = _detile_call(user_table.T, item_table.T)
    return _gather_call(uidx3, iidx3, wb, b16, uflat, iflat)


# speedup vs baseline: 1.0930x; 1.0930x over previous
import functools

import jax
import jax.numpy as jnp
from jax import lax
from jax.experimental import pallas as pl
from jax.experimental.pallas import tpu as pltpu
from jax.experimental.pallas import tpu_sc as plsc

B = 16384
D = 32
L = 16
NC = 2
NS = 16
NW = NC * NS
BPW = B // NW
CHUNK = 128
NCHUNK = BPW // CHUNK
N = 1_000_000
NPAD = 1_000_064
TC_PER_GRP = 13
GRP = 601
NGRP = 4 * GRP
GRP_W = TC_PER_GRP * CHUNK
PER_WORKER = (NGRP + NW - 1) // NW


NBUF = 4


def _detile_body(utab_hbm, itab_hbm, uflat_hbm, iflat_hbm,
                 slab_u0, slab_i0, slab_u1, slab_i1,
                 slab_u2, slab_i2, slab_u3, slab_i3,
                 rd_u, rd_i, wr_u, wr_i):
    wid = lax.axis_index("s") * NC + lax.axis_index("c")
    slabs = ((slab_u0, slab_i0), (slab_u1, slab_i1),
             (slab_u2, slab_i2), (slab_u3, slab_i3))
    dummy_src = utab_hbm.at[pl.ds(0, 8), pl.ds(0, GRP_W)]

    gid0 = wid
    tr0 = gid0 // GRP
    tc00 = (gid0 % GRP) * TC_PER_GRP
    pltpu.async_copy(
        utab_hbm.at[pl.ds(tr0 * 8, 8), pl.ds(tc00 * CHUNK, GRP_W)],
        slabs[0][0], rd_u)
    pltpu.async_copy(
        itab_hbm.at[pl.ds(tr0 * 8, 8), pl.ds(tc00 * CHUNK, GRP_W)],
        slabs[0][1], rd_i)

    def step(k, slot):
        su, si = slabs[slot]
        pu, pi = slabs[(slot + 1) % NBUF]
        gid = wid + k * NW

        @pl.when(gid < NGRP)
        def _():
            tr = gid // GRP
            tc0 = (gid % GRP) * TC_PER_GRP

            @pl.when(k + 1 >= NBUF)
            def _():
                pltpu.make_async_copy(dummy_src, pu, wr_u).wait()
                pltpu.make_async_copy(dummy_src, pi, wr_i).wait()

            gid2 = gid + NW

            @pl.when(gid2 < NGRP)
            def _():
                tr2 = gid2 // GRP
                tc2 = (gid2 % GRP) * TC_PER_GRP
                pltpu.async_copy(
                    utab_hbm.at[pl.ds(tr2 * 8, 8), pl.ds(tc2 * CHUNK, GRP_W)],
                    pu, rd_u)
                pltpu.async_copy(
                    itab_hbm.at[pl.ds(tr2 * 8, 8), pl.ds(tc2 * CHUNK, GRP_W)],
                    pi, rd_i)

            pltpu.make_async_copy(dummy_src, su, rd_u).wait()
            pltpu.make_async_copy(dummy_src, si, rd_i).wait()
            for dr in range(8):
                off = (tr * 8 + dr) * NPAD + tc0 * CHUNK
                pltpu.async_copy(su.at[dr], uflat_hbm.at[pl.ds(off, GRP_W)], wr_u)
                pltpu.async_copy(si.at[dr], iflat_hbm.at[pl.ds(off, GRP_W)], wr_i)

    def body(kk, carry):
        for b2 in range(NBUF):
            step(NBUF * kk + b2, b2)
        return carry

    lax.fori_loop(0, (PER_WORKER + NBUF - 1) // NBUF, body, 0)

    for b2 in range(NBUF - 1):
        su, si = slabs[b2]
        pltpu.make_async_copy(dummy_src, su, wr_u).wait()
        pltpu.make_async_copy(dummy_src, si, wr_i).wait()


def _gather_body(uidx_hbm, iidx_hbm, w_hbm, b_hbm, uflat_hbm, iflat_hbm,
                 out_hbm, uidx_v, iidx_v, uaddr_v, iaddr_v, urows_v, irows_v,
                 w_v, b_v, out_v, sem_u, sem_i):
    wid = lax.axis_index("s") * NC + lax.axis_index("c")
    base = wid * BPW

    pltpu.sync_copy(uidx_hbm.at[wid], uidx_v)
    pltpu.sync_copy(iidx_hbm.at[wid], iidx_v)
    pltpu.sync_copy(w_hbm, w_v)
    pltpu.sync_copy(b_hbm, b_v)

    for c in range(NCHUNK):
        for j in range(CHUNK // L):
            sl = pl.ds(c * CHUNK + j * L, L)
            uiv = uidx_v[c, pl.ds(j * L, L)]
            iiv = iidx_v[c, pl.ds(j * L, L)]
            for d in range(D):
                uaddr_v[d, sl] = uiv + d * NPAD
                iaddr_v[d, sl] = iiv + d * NPAD

    copies = []
    for d in range(D):
        copies.append(pltpu.async_copy(
            uflat_hbm.at[uaddr_v.at[d]], urows_v.at[d], sem_u))
        copies.append(pltpu.async_copy(
            iflat_hbm.at[iaddr_v.at[d]], irows_v.at[d], sem_i))
    for cp in copies:
        cp.wait()

    bvec = b_v[...]
    wvecs = [w_v[d] for d in range(D)]

    def body(g, carry):
        acc = bvec
        for d in range(D):
            uu = urows_v[d, pl.ds(g * L, L)]
            ii = irows_v[d, pl.ds(g * L, L)]
            acc = acc + uu * ii * wvecs[d]
        out_v[pl.ds(g * L, L)] = acc
        return carry

    lax.fori_loop(0, BPW // L, body, 0)
    pltpu.sync_copy(out_v, out_hbm.at[pl.ds(base, BPW)])


def _detile_call(utabT, itabT):
    mesh = plsc.VectorSubcoreMesh(core_axis_name="c", subcore_axis_name="s")
    kern = functools.partial(
        pl.kernel,
        mesh=mesh,
        compiler_params=pltpu.CompilerParams(
            needs_layout_passes=False, use_tc_tiling_on_sc=True),
        out_type=(jax.ShapeDtypeStruct((D * NPAD,), jnp.float32),
                  jax.ShapeDtypeStruct((D * NPAD,), jnp.float32)),
        scratch_types=[
            pltpu.VMEM((8, GRP_W), jnp.float32),
            pltpu.VMEM((8, GRP_W), jnp.float32),
            pltpu.VMEM((8, GRP_W), jnp.float32),
            pltpu.VMEM((8, GRP_W), jnp.float32),
            pltpu.VMEM((8, GRP_W), jnp.float32),
            pltpu.VMEM((8, GRP_W), jnp.float32),
            pltpu.VMEM((8, GRP_W), jnp.float32),
            pltpu.VMEM((8, GRP_W), jnp.float32),
            pltpu.SemaphoreType.DMA,
            pltpu.SemaphoreType.DMA,
            pltpu.SemaphoreType.DMA,
            pltpu.SemaphoreType.DMA,
        ],
    )(_detile_body)
    return kern(utabT, itabT)


def _gather_call(uidx3, iidx3, wb, b16, uflat, iflat):
    mesh = plsc.VectorSubcoreMesh(core_axis_name="c", subcore_axis_name="s")
    kern = functools.partial(
        pl.kernel,
        mesh=mesh,
        compiler_params=pltpu.CompilerParams(
            needs_layout_passes=False, use_tc_tiling_on_sc=False),
        out_type=jax.ShapeDtypeStruct((B,), jnp.float32),
        scratch_types=[
            pltpu.VMEM((NCHUNK, CHUNK), jnp.int32),
            pltpu.VMEM((NCHUNK, CHUNK), jnp.int32),
            pltpu.VMEM((D, BPW), jnp.int32),
            pltpu.VMEM((D, BPW), jnp.int32),
            pltpu.VMEM((D, BPW), jnp.float32),
            pltpu.VMEM((D, BPW), jnp.float32),
            pltpu.VMEM((D, L), jnp.float32),
            pltpu.VMEM((L,), jnp.float32),
            pltpu.VMEM((BPW,), jnp.float32),
            pltpu.SemaphoreType.DMA,
            pltpu.SemaphoreType.DMA,
        ],
    )(_gather_body)
    return kern(uidx3, iidx3, wb, b16, uflat, iflat)


def kernel(user_indices, item_indices, ratings, user_table, item_table, W, b):
    del ratings
    uidx3 = user_indices.reshape(NW, NCHUNK, CHUNK)
    iidx3 = item_indices.reshape(NW, NCHUNK, CHUNK)
    wb = jnp.broadcast_to(W, (D, L))
    b16 = jnp.broadcast_to(b, (L,))
    uflat, iflat = _detile_call(user_table.T, item_table.T)
    return _gather_call(uidx3, iidx3, wb, b16, uflat, iflat)


# speedup vs baseline: 1.0973x; 1.0039x over previous
import functools

import jax
import jax.numpy as jnp
from jax import lax
from jax.experimental import pallas as pl
from jax.experimental.pallas import tpu as pltpu
from jax.experimental.pallas import tpu_sc as plsc

B = 16384
D = 32
L = 16
NC = 2
NS = 16
NW = NC * NS
BPW = B // NW
CHUNK = 128
NCHUNK = BPW // CHUNK
N = 1_000_000
NPAD = 1_000_064
TC_PER_GRP = 13
GRP = 601
NGRP = 4 * GRP
GRP_W = TC_PER_GRP * CHUNK
PER_WORKER = (NGRP + NW - 1) // NW


NBUF = 4


def _detile_body(utab_hbm, itab_hbm, uflat_hbm, iflat_hbm,
                 slab_u0, slab_i0, slab_u1, slab_i1,
                 slab_u2, slab_i2, slab_u3, slab_i3,
                 rd_u, rd_i, wr_u, wr_i):
    wid = lax.axis_index("s") * NC + lax.axis_index("c")
    slabs = ((slab_u0, slab_i0), (slab_u1, slab_i1),
             (slab_u2, slab_i2), (slab_u3, slab_i3))
    dummy_src = utab_hbm.at[pl.ds(0, 8), pl.ds(0, GRP_W)]

    gid0 = wid
    tr0 = gid0 // GRP
    tc00 = (gid0 % GRP) * TC_PER_GRP
    pltpu.async_copy(
        utab_hbm.at[pl.ds(tr0 * 8, 8), pl.ds(tc00 * CHUNK, GRP_W)],
        slabs[0][0], rd_u)
    pltpu.async_copy(
        itab_hbm.at[pl.ds(tr0 * 8, 8), pl.ds(tc00 * CHUNK, GRP_W)],
        slabs[0][1], rd_i)

    def step(k, slot):
        su, si = slabs[slot]
        pu, pi = slabs[(slot + 1) % NBUF]
        gid = wid + k * NW

        @pl.when(gid < NGRP)
        def _():
            tr = gid // GRP
            tc0 = (gid % GRP) * TC_PER_GRP

            @pl.when(k + 1 >= NBUF)
            def _():
                pltpu.make_async_copy(dummy_src, pu, wr_u).wait()
                pltpu.make_async_copy(dummy_src, pi, wr_i).wait()

            gid2 = gid + NW

            @pl.when(gid2 < NGRP)
            def _():
                tr2 = gid2 // GRP
                tc2 = (gid2 % GRP) * TC_PER_GRP
                pltpu.async_copy(
                    utab_hbm.at[pl.ds(tr2 * 8, 8), pl.ds(tc2 * CHUNK, GRP_W)],
                    pu, rd_u)
                pltpu.async_copy(
                    itab_hbm.at[pl.ds(tr2 * 8, 8), pl.ds(tc2 * CHUNK, GRP_W)],
                    pi, rd_i)

            pltpu.make_async_copy(dummy_src, su, rd_u).wait()
            pltpu.make_async_copy(dummy_src, si, rd_i).wait()
            for dr in range(8):
                off = (tr * 8 + dr) * NPAD + tc0 * CHUNK
                pltpu.async_copy(su.at[dr], uflat_hbm.at[pl.ds(off, GRP_W)], wr_u)
                pltpu.async_copy(si.at[dr], iflat_hbm.at[pl.ds(off, GRP_W)], wr_i)

    def body(kk, carry):
        for b2 in range(NBUF):
            step(NBUF * kk + b2, b2)
        return carry

    lax.fori_loop(0, (PER_WORKER + NBUF - 1) // NBUF, body, 0)

    for b2 in range(NBUF - 1):
        su, si = slabs[b2]
        pltpu.make_async_copy(dummy_src, su, wr_u).wait()
        pltpu.make_async_copy(dummy_src, si, wr_i).wait()


def _gather_body(uidx_hbm, iidx_hbm, w_hbm, b_hbm, uflat_hbm, iflat_hbm,
                 out_hbm, uidx_v, iidx_v, uaddr_v, iaddr_v, urows_v, irows_v,
                 w_v, b_v, out_v, sem_u, sem_i):
    wid = lax.axis_index("s") * NC + lax.axis_index("c")
    base = wid * BPW

    pltpu.sync_copy(uidx_hbm.at[wid], uidx_v)
    pltpu.sync_copy(iidx_hbm.at[wid], iidx_v)
    pltpu.sync_copy(w_hbm, w_v)
    pltpu.sync_copy(b_hbm, b_v)

    copies = []
    for d in range(D):
        for c in range(NCHUNK):
            for j in range(CHUNK // L):
                sl = pl.ds(c * CHUNK + j * L, L)
                uaddr_v[d, sl] = uidx_v[c, pl.ds(j * L, L)] + d * NPAD
                iaddr_v[d, sl] = iidx_v[c, pl.ds(j * L, L)] + d * NPAD
        copies.append(pltpu.async_copy(
            uflat_hbm.at[uaddr_v.at[d]], urows_v.at[d], sem_u))
        copies.append(pltpu.async_copy(
            iflat_hbm.at[iaddr_v.at[d]], irows_v.at[d], sem_i))
    for cp in copies:
        cp.wait()

    bvec = b_v[...]
    wvecs = [w_v[d] for d in range(D)]

    def body(g, carry):
        acc = bvec
        for d in range(D):
            uu = urows_v[d, pl.ds(g * L, L)]
            ii = irows_v[d, pl.ds(g * L, L)]
            acc = acc + uu * ii * wvecs[d]
        out_v[pl.ds(g * L, L)] = acc
        return carry

    lax.fori_loop(0, BPW // L, body, 0)
    pltpu.sync_copy(out_v, out_hbm.at[pl.ds(base, BPW)])


def _detile_call(utabT, itabT):
    mesh = plsc.VectorSubcoreMesh(core_axis_name="c", subcore_axis_name="s")
    kern = functools.partial(
        pl.kernel,
        mesh=mesh,
        compiler_params=pltpu.CompilerParams(
            needs_layout_passes=False, use_tc_tiling_on_sc=True),
        out_type=(jax.ShapeDtypeStruct((D * NPAD,), jnp.float32),
                  jax.ShapeDtypeStruct((D * NPAD,), jnp.float32)),
        scratch_types=[
            pltpu.VMEM((8, GRP_W), jnp.float32),
            pltpu.VMEM((8, GRP_W), jnp.float32),
            pltpu.VMEM((8, GRP_W), jnp.float32),
            pltpu.VMEM((8, GRP_W), jnp.float32),
            pltpu.VMEM((8, GRP_W), jnp.float32),
            pltpu.VMEM((8, GRP_W), jnp.float32),
            pltpu.VMEM((8, GRP_W), jnp.float32),
            pltpu.VMEM((8, GRP_W), jnp.float32),
            pltpu.SemaphoreType.DMA,
            pltpu.SemaphoreType.DMA,
            pltpu.SemaphoreType.DMA,
            pltpu.SemaphoreType.DMA,
        ],
    )(_detile_body)
    return kern(utabT, itabT)


def _gather_call(uidx3, iidx3, wb, b16, uflat, iflat):
    mesh = plsc.VectorSubcoreMesh(core_axis_name="c", subcore_axis_name="s")
    kern = functools.partial(
        pl.kernel,
        mesh=mesh,
        compiler_params=pltpu.CompilerParams(
            needs_layout_passes=False, use_tc_tiling_on_sc=False),
        out_type=jax.ShapeDtypeStruct((B,), jnp.float32),
        scratch_types=[
            pltpu.VMEM((NCHUNK, CHUNK), jnp.int32),
            pltpu.VMEM((NCHUNK, CHUNK), jnp.int32),
            pltpu.VMEM((D, BPW), jnp.int32),
            pltpu.VMEM((D, BPW), jnp.int32),
            pltpu.VMEM((D, BPW), jnp.float32),
            pltpu.VMEM((D, BPW), jnp.float32),
            pltpu.VMEM((D, L), jnp.float32),
            pltpu.VMEM((L,), jnp.float32),
            pltpu.VMEM((BPW,), jnp.float32),
            pltpu.SemaphoreType.DMA,
            pltpu.SemaphoreType.DMA,
        ],
    )(_gather_body)
    return kern(uidx3, iidx3, wb, b16, uflat, iflat)


def kernel(user_indices, item_indices, ratings, user_table, item_table, W, b):
    del ratings
    uidx3 = user_indices.reshape(NW, NCHUNK, CHUNK)
    iidx3 = item_indices.reshape(NW, NCHUNK, CHUNK)
    wb = jnp.broadcast_to(W, (D, L))
    b16 = jnp.broadcast_to(b, (L,))
    uflat, iflat = _detile_call(user_table.T, item_table.T)
    return _gather_call(uidx3, iidx3, wb, b16, uflat, iflat)
